# baseline (speedup 1.0000x reference)
import jax
import jax.numpy as jnp
from jax import lax
from jax.experimental import pallas as pl
from jax.experimental.pallas import tpu as pltpu
from jax.experimental.pallas import tpu_sc as plsc

N = 10000
E = 320000
D = 128

NC = 2
NS = 16
NW = NC * NS

CHUNK = 128
EPW = 10240
NCHUNK = EPW // CHUNK
EP = EPW * NW
NPAD = 10240
ROWS_PER_TILE = NPAD // NS


def _mesh():
    return plsc.VectorSubcoreMesh(
        core_axis_name="c", subcore_axis_name="s", num_cores=NC, num_subcores=NS
    )


def _deg_body(cols_hbm, out_hbm, col_v, ones_v, deg_sh, zrow_v):
    cid = lax.axis_index("c")
    sid = lax.axis_index("s")
    wid = sid * NC + cid

    zero16 = jnp.zeros((16,), jnp.float32)
    one16 = jnp.ones((16,), jnp.float32)
    for k in range(CHUNK // 16):
        ones_v[pl.ds(k * 16, 16)] = one16

    def zfill(i, _):
        zrow_v[pl.ds(i * 16, 16)] = zero16
        return 0
    lax.fori_loop(0, ROWS_PER_TILE // 16, zfill, 0)

    pltpu.sync_copy(zrow_v, deg_sh.at[pl.ds(sid * ROWS_PER_TILE, ROWS_PER_TILE)])
    plsc.subcore_barrier()

    pltpu.sync_copy(cols_hbm.at[wid], col_v)

    def body(j, _):
        pltpu.sync_copy(ones_v, deg_sh.at[col_v.at[j]], add=True)
        return 0
    lax.fori_loop(0, NCHUNK, body, 0)
    plsc.subcore_barrier()

    pltpu.sync_copy(
        deg_sh.at[pl.ds(sid * ROWS_PER_TILE, ROWS_PER_TILE)],
        out_hbm.at[cid, pl.ds(sid * ROWS_PER_TILE, ROWS_PER_TILE)],
    )


@jax.jit
def _sc_deg(cols3):
    return pl.kernel(
        _deg_body,
        out_type=jax.ShapeDtypeStruct((NC, NPAD), jnp.float32),
        mesh=_mesh(),
        scratch_types=[
            pltpu.VMEM((NCHUNK, CHUNK), jnp.int32),
            pltpu.VMEM((CHUNK,), jnp.float32),
            pltpu.VMEM_SHARED((NPAD,), jnp.float32),
            pltpu.VMEM((ROWS_PER_TILE,), jnp.float32),
        ],
    )(cols3)


def _hop_body(hs_hbm, pk_hbm, out_hbm,
              pk_v, row_u, col_u, gbuf0, gbuf1, zblk_v, acc_sh, sem0, sem1):
    cid = lax.axis_index("c")
    sid = lax.axis_index("s")
    wid = sid * NC + cid

    zero16 = jnp.zeros((16,), jnp.float32)

    def zfill(i, _):
        for k in range(D // 16):
            zblk_v[i, pl.ds(k * 16, 16)] = zero16
        return 0
    lax.fori_loop(0, 16, zfill, 0)

    def zcopy(i, _):
        pltpu.async_copy(
            zblk_v, acc_sh.at[pl.ds(sid * ROWS_PER_TILE + i * 16, 16)], sem0
        )
        return 0
    lax.fori_loop(0, ROWS_PER_TILE // 16, zcopy, 0)

    def zdrain(i, _):
        pltpu.make_async_copy(
            zblk_v, acc_sh.at[pl.ds(sid * ROWS_PER_TILE + i * 16, 16)], sem0
        ).wait()
        return 0
    lax.fori_loop(0, ROWS_PER_TILE // 16, zdrain, 0)
    plsc.subcore_barrier()

    pltpu.sync_copy(pk_hbm.at[wid], pk_v)

    def unpack(j, s):
        for k in range(CHUNK // 16):
            p = pk_v[j, pl.ds(k * 16, 16)]
            row_u[s, pl.ds(k * 16, 16)] = jnp.bitwise_and(p, 0xFFFF)
            col_u[s, pl.ds(k * 16, 16)] = jnp.right_shift(p, 16)
        return 0

    unpack(0, 0)
    pltpu.async_copy(hs_hbm.at[row_u.at[0]], gbuf0, sem0)

    def body(j, _):
        even = lax.rem(j, 2) == 0

        @pl.when(even)
        def _():
            @pl.when(j + 1 < NCHUNK)
            def _():
                unpack(j + 1, 1)
                pltpu.async_copy(hs_hbm.at[row_u.at[1]], gbuf1, sem1)
            pltpu.make_async_copy(hs_hbm.at[row_u.at[0]], gbuf0, sem0).wait()
            pltpu.sync_copy(gbuf0, acc_sh.at[col_u.at[0]], add=True)

        @pl.when(jnp.logical_not(even))
        def _():
            @pl.when(j + 1 < NCHUNK)
            def _():
                unpack(j + 1, 0)
                pltpu.async_copy(hs_hbm.at[row_u.at[0]], gbuf0, sem0)
            pltpu.make_async_copy(hs_hbm.at[row_u.at[1]], gbuf1, sem1).wait()
            pltpu.sync_copy(gbuf1, acc_sh.at[col_u.at[1]], add=True)

        return 0

    lax.fori_loop(0, NCHUNK, body, 0)
    plsc.subcore_barrier()

    pltpu.sync_copy(
        acc_sh.at[pl.ds(sid * ROWS_PER_TILE, ROWS_PER_TILE)],
        out_hbm.at[cid, pl.ds(sid * ROWS_PER_TILE, ROWS_PER_TILE)],
    )


@jax.jit
def _sc_hop(hs, pk3):
    return pl.kernel(
        _hop_body,
        out_type=jax.ShapeDtypeStruct((NC, NPAD, D), jnp.float32),
        mesh=_mesh(),
        scratch_types=[
            pltpu.VMEM((NCHUNK, CHUNK), jnp.int32),
            pltpu.VMEM((2, CHUNK), jnp.int32),
            pltpu.VMEM((2, CHUNK), jnp.int32),
            pltpu.VMEM((CHUNK, D), jnp.float32),
            pltpu.VMEM((CHUNK, D), jnp.float32),
            pltpu.VMEM((16, D), jnp.float32),
            pltpu.VMEM_SHARED((NPAD, D), jnp.float32),
            pltpu.SemaphoreType.DMA,
            pltpu.SemaphoreType.DMA,
        ],
    )(hs, pk3)


def _hop2m_body(g1_hbm, hs1_hbm, pk_hbm, degp_hbm, hs2_hbm, out_hbm,
                pk_v, row_u, col_u, gbuf0, gbuf1, zblk_v, dv0, dv1, d2buf,
                acc_sh, sem0, sem1):
    cid = lax.axis_index("c")
    sid = lax.axis_index("s")
    wid = sid * NC + cid
    base = sid * ROWS_PER_TILE

    pltpu.sync_copy(degp_hbm.at[0, pl.ds(base, ROWS_PER_TILE)], dv0)
    pltpu.sync_copy(degp_hbm.at[1, pl.ds(base, ROWS_PER_TILE)], dv1)

    def sub(i, _):
        off = base + i * 64
        pltpu.sync_copy(g1_hbm.at[0, pl.ds(off, 64)], gbuf0.at[pl.ds(0, 64)])
        pltpu.sync_copy(g1_hbm.at[1, pl.ds(off, 64)], gbuf0.at[pl.ds(64, 64)])
        pltpu.sync_copy(hs1_hbm.at[pl.ds(off, 64)], gbuf1.at[pl.ds(0, 64)])
        for q in range(4):
            d0 = dv0[pl.ds(i * 64 + q * 16, 16)]
            d1 = dv1[pl.ds(i * 64 + q * 16, 16)]
            d2buf[pl.ds(0, 16)] = 1.0 / (d0 + d1 + 1.0)
            for r in range(16):
                rr = q * 16 + r
                d2 = d2buf[pl.ds(r, 16)][0]
                for k in range(D // 16):
                    sl = pl.ds(k * 16, 16)
                    gbuf1[64 + rr, sl] = (
                        gbuf0[rr, sl] + gbuf0[64 + rr, sl] + gbuf1[rr, sl]
                    ) * d2
        pltpu.sync_copy(gbuf1.at[pl.ds(64, 64)], hs2_hbm.at[cid, pl.ds(off, 64)])
        return 0
    lax.fori_loop(0, ROWS_PER_TILE // 64, sub, 0)

    zero16 = jnp.zeros((16,), jnp.float32)

    def zfill(i, _):
        for k in range(D // 16):
            zblk_v[i, pl.ds(k * 16, 16)] = zero16
        return 0
    lax.fori_loop(0, 16, zfill, 0)

    def zcopy(i, _):
        pltpu.async_copy(zblk_v, acc_sh.at[pl.ds(base + i * 16, 16)], sem0)
        return 0
    lax.fori_loop(0, ROWS_PER_TILE // 16, zcopy, 0)

    def zdrain(i, _):
        pltpu.make_async_copy(
            zblk_v, acc_sh.at[pl.ds(base + i * 16, 16)], sem0).wait()
        return 0
    lax.fori_loop(0, ROWS_PER_TILE // 16, zdrain, 0)
    plsc.subcore_barrier()

    pltpu.sync_copy(pk_hbm.at[wid], pk_v)
    table = hs2_hbm.at[cid]

    def unpack(j, s):
        for k in range(CHUNK // 16):
            p = pk_v[j, pl.ds(k * 16, 16)]
            row_u[s, pl.ds(k * 16, 16)] = jnp.bitwise_and(p, 0xFFFF)
            col_u[s, pl.ds(k * 16, 16)] = jnp.right_shift(p, 16)
        return 0

    unpack(0, 0)
    pltpu.async_copy(table.at[row_u.at[0]], gbuf0, sem0)

    def body(j, _):
        even = lax.rem(j, 2) == 0

        @pl.when(even)
        def _():
            @pl.when(j + 1 < NCHUNK)
            def _():
                unpack(j + 1, 1)
                pltpu.async_copy(table.at[row_u.at[1]], gbuf1, sem1)
            pltpu.make_async_copy(table.at[row_u.at[0]], gbuf0, sem0).wait()
            pltpu.sync_copy(gbuf0, acc_sh.at[col_u.at[0]], add=True)

        @pl.when(jnp.logical_not(even))
        def _():
            @pl.when(j + 1 < NCHUNK)
            def _():
                unpack(j + 1, 0)
                pltpu.async_copy(table.at[row_u.at[0]], gbuf0, sem0)
            pltpu.make_async_copy(table.at[row_u.at[1]], gbuf1, sem1).wait()
            pltpu.sync_copy(gbuf1, acc_sh.at[col_u.at[1]], add=True)

        return 0

    lax.fori_loop(0, NCHUNK, body, 0)
    plsc.subcore_barrier()

    pltpu.sync_copy(
        acc_sh.at[pl.ds(base, ROWS_PER_TILE)],
        out_hbm.at[cid, pl.ds(base, ROWS_PER_TILE)],
    )


@jax.jit
def _sc_hop2m(g1, hs1, pk3, degp_full):
    return pl.kernel(
        _hop2m_body,
        out_type=(
            jax.ShapeDtypeStruct((NC, NPAD, D), jnp.float32),
            jax.ShapeDtypeStruct((NC, NPAD, D), jnp.float32),
        ),
        mesh=_mesh(),
        scratch_types=[
            pltpu.VMEM((NCHUNK, CHUNK), jnp.int32),
            pltpu.VMEM((2, CHUNK), jnp.int32),
            pltpu.VMEM((2, CHUNK), jnp.int32),
            pltpu.VMEM((CHUNK, D), jnp.float32),
            pltpu.VMEM((CHUNK, D), jnp.float32),
            pltpu.VMEM((16, D), jnp.float32),
            pltpu.VMEM((ROWS_PER_TILE,), jnp.float32),
            pltpu.VMEM((ROWS_PER_TILE,), jnp.float32),
            pltpu.VMEM((32,), jnp.float32),
            pltpu.VMEM_SHARED((NPAD, D), jnp.float32),
            pltpu.SemaphoreType.DMA,
            pltpu.SemaphoreType.DMA,
        ],
    )(g1, hs1, pk3, degp_full)


ROWB = 1000
ROWB_S = 1024


def _dis_block(degp_blk):
    deg = degp_blk[:, 0:1] + degp_blk[:, 1:2] + 1.0
    return lax.rsqrt(deg)


def _mm_body(x_ref, w_ref, b_ref, out_ref):
    out_ref[...] = lax.dot_general(
        x_ref[...], w_ref[...], (((1,), (1,)), ((), ())),
        preferred_element_type=jnp.float32,
    ) + b_ref[...]


def _tc_mm(x, W, b):
    return pl.pallas_call(
        _mm_body,
        grid=(N // ROWB,),
        in_specs=[
            pl.BlockSpec((ROWB, D), lambda i: (i, 0)),
            pl.BlockSpec((D, D), lambda i: (0, 0)),
            pl.BlockSpec((1, D), lambda i: (0, 0)),
        ],
        out_specs=pl.BlockSpec((ROWB, D), lambda i: (i, 0)),
        out_shape=jax.ShapeDtypeStruct((N, D), jnp.float32),
    )(x, W, b.reshape(1, D))


def _scale_body(h_ref, degp_ref, out_ref):
    dis = _dis_block(degp_ref[...])
    out_ref[...] = dis * h_ref[...]


def _tc_scale(hpad, degp):
    return pl.pallas_call(
        _scale_body,
        grid=(NPAD // ROWB_S,),
        in_specs=[
            pl.BlockSpec((ROWB_S, D), lambda i: (i, 0)),
            pl.BlockSpec((ROWB_S, 2), lambda i: (i, 0)),
        ],
        out_specs=pl.BlockSpec((ROWB_S, D), lambda i: (i, 0)),
        out_shape=jax.ShapeDtypeStruct((NPAD, D), jnp.float32),
    )(hpad, degp)


def _fin_body(g_ref, hs2_ref, degp_ref, out_ref):
    dis = _dis_block(degp_ref[...])
    h = dis * (g_ref[0] + g_ref[1] + hs2_ref[0])
    m = jnp.max(h, axis=1, keepdims=True)
    z = h - m
    lse = jnp.log(jnp.sum(jnp.exp(z), axis=1, keepdims=True))
    out_ref[...] = z - lse


def _tc_fin(g2, hs2c, degp):
    return pl.pallas_call(
        _fin_body,
        grid=(N // ROWB,),
        in_specs=[
            pl.BlockSpec((2, ROWB, D), lambda i: (0, i, 0)),
            pl.BlockSpec((1, ROWB, D), lambda i: (0, i, 0)),
            pl.BlockSpec((ROWB, 2), lambda i: (i, 0)),
        ],
        out_specs=pl.BlockSpec((ROWB, D), lambda i: (i, 0)),
        out_shape=jax.ShapeDtypeStruct((N, D), jnp.float32),
    )(g2, hs2c, degp)


def kernel(x, edge_index, W, b):
    row = edge_index[0]
    col = edge_index[1]
    pad = EP - E
    arp = jnp.arange(pad, dtype=jnp.int32)
    rowp = jnp.concatenate([row, arp % N])
    colp = jnp.concatenate([col, N + arp % (NPAD - N)])
    cols3 = colp.reshape(NW, NCHUNK, CHUNK)
    pk3 = (rowp | (colp << 16)).reshape(NW, NCHUNK, CHUNK)

    h = _tc_mm(x, W, b)
    degp_full = _sc_deg(cols3)
    degp = degp_full.T

    hpad = jnp.pad(h, ((0, NPAD - N), (0, 0)))
    hs1 = _tc_scale(hpad, degp)
    g1 = _sc_hop(hs1, pk3)
    hs2c, g2 = _sc_hop2m(g1, hs1, pk3, degp_full)
    return _tc_fin(g2, hs2c, degp)


# speedup vs baseline: 1.1806x; 1.1806x over previous
import jax
import jax.numpy as jnp
from jax import lax
from jax.experimental import pallas as pl
from jax.experimental.pallas import tpu as pltpu
from jax.experimental.pallas import tpu_sc as plsc

N = 10000
E = 320000
D = 128

NC = 2
NS = 16
NW = NC * NS

CHUNK = 128
EPW = 10240
NCHUNK = EPW // CHUNK
EP = EPW * NW
NPAD = 10240
ROWS_PER_TILE = NPAD // NS


def _mesh():
    return plsc.VectorSubcoreMesh(
        core_axis_name="c", subcore_axis_name="s", num_cores=NC, num_subcores=NS
    )


def _deg_body(cols_hbm, out_hbm, col_v, ones_v, deg_sh, zrow_v, semd):
    cid = lax.axis_index("c")
    sid = lax.axis_index("s")
    wid = sid * NC + cid

    zero16 = jnp.zeros((16,), jnp.float32)
    one16 = jnp.ones((16,), jnp.float32)
    for k in range(CHUNK // 16):
        ones_v[pl.ds(k * 16, 16)] = one16

    def zfill(i, _):
        zrow_v[pl.ds(i * 16, 16)] = zero16
        return 0
    lax.fori_loop(0, ROWS_PER_TILE // 16, zfill, 0)

    pltpu.sync_copy(zrow_v, deg_sh.at[pl.ds(sid * ROWS_PER_TILE, ROWS_PER_TILE)])
    plsc.subcore_barrier()

    pltpu.sync_copy(cols_hbm.at[wid], col_v)

    def body(j, _):
        pltpu.async_copy(ones_v, deg_sh.at[col_v.at[j]], semd, add=True)
        return 0
    lax.fori_loop(0, NCHUNK, body, 0)

    def drain(j, _):
        pltpu.make_async_copy(ones_v, deg_sh.at[col_v.at[j]], semd).wait()
        return 0
    lax.fori_loop(0, NCHUNK, drain, 0)
    plsc.subcore_barrier()

    pltpu.sync_copy(
        deg_sh.at[pl.ds(sid * ROWS_PER_TILE, ROWS_PER_TILE)],
        out_hbm.at[cid, pl.ds(sid * ROWS_PER_TILE, ROWS_PER_TILE)],
    )


@jax.jit
def _sc_deg(cols3):
    return pl.kernel(
        _deg_body,
        out_type=jax.ShapeDtypeStruct((NC, NPAD), jnp.float32),
        mesh=_mesh(),
        scratch_types=[
            pltpu.VMEM((NCHUNK, CHUNK), jnp.int32),
            pltpu.VMEM((CHUNK,), jnp.float32),
            pltpu.VMEM_SHARED((NPAD,), jnp.float32),
            pltpu.VMEM((ROWS_PER_TILE,), jnp.float32),
            pltpu.SemaphoreType.DMA,
        ],
    )(cols3)


def _hop_body(hs_hbm, pk_hbm, out_hbm,
              pk_v, row_u, col_u, gbuf0, gbuf1, zblk_v, acc_sh, sem0, sem1):
    cid = lax.axis_index("c")
    sid = lax.axis_index("s")
    wid = sid * NC + cid

    zero16 = jnp.zeros((16,), jnp.float32)

    def zfill(i, _):
        for k in range(D // 16):
            zblk_v[i, pl.ds(k * 16, 16)] = zero16
        return 0
    lax.fori_loop(0, 16, zfill, 0)

    def zcopy(i, _):
        pltpu.async_copy(
            zblk_v, acc_sh.at[pl.ds(sid * ROWS_PER_TILE + i * 16, 16)], sem0
        )
        return 0
    lax.fori_loop(0, ROWS_PER_TILE // 16, zcopy, 0)

    def zdrain(i, _):
        pltpu.make_async_copy(
            zblk_v, acc_sh.at[pl.ds(sid * ROWS_PER_TILE + i * 16, 16)], sem0
        ).wait()
        return 0
    lax.fori_loop(0, ROWS_PER_TILE // 16, zdrain, 0)
    plsc.subcore_barrier()

    pltpu.sync_copy(pk_hbm.at[wid], pk_v)

    def unpack(j, s):
        for k in range(CHUNK // 16):
            p = pk_v[j, pl.ds(k * 16, 16)]
            row_u[s, pl.ds(k * 16, 16)] = jnp.bitwise_and(p, 0xFFFF)
            col_u[s, pl.ds(k * 16, 16)] = jnp.right_shift(p, 16)
        return 0

    unpack(0, 0)
    pltpu.async_copy(hs_hbm.at[row_u.at[0]], gbuf0, sem0)

    def pair(p, _):
        unpack(2 * p + 1, 1)
        pltpu.async_copy(hs_hbm.at[row_u.at[1]], gbuf1, sem1)
        pltpu.make_async_copy(hs_hbm.at[row_u.at[0]], gbuf0, sem0).wait()
        pltpu.sync_copy(gbuf0, acc_sh.at[col_u.at[0]], add=True)

        @pl.when(p + 1 < NCHUNK // 2)
        def _():
            unpack(2 * p + 2, 0)
            pltpu.async_copy(hs_hbm.at[row_u.at[0]], gbuf0, sem0)
        pltpu.make_async_copy(hs_hbm.at[row_u.at[1]], gbuf1, sem1).wait()
        pltpu.sync_copy(gbuf1, acc_sh.at[col_u.at[1]], add=True)
        return 0

    lax.fori_loop(0, NCHUNK // 2, pair, 0)
    plsc.subcore_barrier()

    pltpu.sync_copy(
        acc_sh.at[pl.ds(sid * ROWS_PER_TILE, ROWS_PER_TILE)],
        out_hbm.at[cid, pl.ds(sid * ROWS_PER_TILE, ROWS_PER_TILE)],
    )


@jax.jit
def _sc_hop(hs, pk3):
    return pl.kernel(
        _hop_body,
        out_type=jax.ShapeDtypeStruct((NC, NPAD, D), jnp.float32),
        mesh=_mesh(),
        scratch_types=[
            pltpu.VMEM((NCHUNK, CHUNK), jnp.int32),
            pltpu.VMEM((2, CHUNK), jnp.int32),
            pltpu.VMEM((2, CHUNK), jnp.int32),
            pltpu.VMEM((CHUNK, D), jnp.float32),
            pltpu.VMEM((CHUNK, D), jnp.float32),
            pltpu.VMEM((16, D), jnp.float32),
            pltpu.VMEM_SHARED((NPAD, D), jnp.float32),
            pltpu.SemaphoreType.DMA,
            pltpu.SemaphoreType.DMA,
        ],
    )(hs, pk3)


ROWB = 1000


def _dis_block(degp_blk):
    deg = degp_blk[:, 0:1] + degp_blk[:, 1:2] + 1.0
    return lax.rsqrt(deg)


def _mm_body(x_ref, w_ref, b_ref, out_ref):
    out_ref[...] = lax.dot_general(
        x_ref[...], w_ref[...], (((1,), (1,)), ((), ())),
        preferred_element_type=jnp.float32,
    ) + b_ref[...]


def _tc_mm(x, W, b):
    return pl.pallas_call(
        _mm_body,
        grid=(N // ROWB,),
        in_specs=[
            pl.BlockSpec((ROWB, D), lambda i: (i, 0)),
            pl.BlockSpec((D, D), lambda i: (0, 0)),
            pl.BlockSpec((1, D), lambda i: (0, 0)),
        ],
        out_specs=pl.BlockSpec((ROWB, D), lambda i: (i, 0)),
        out_shape=jax.ShapeDtypeStruct((N, D), jnp.float32),
    )(x, W, b.reshape(1, D))


def _scale_body(h_ref, degp_ref, out_ref):
    dis = _dis_block(degp_ref[...])
    out_ref[...] = dis * h_ref[...]


def _tc_scale(h, degp):
    return pl.pallas_call(
        _scale_body,
        grid=(N // ROWB,),
        in_specs=[
            pl.BlockSpec((ROWB, D), lambda i: (i, 0)),
            pl.BlockSpec((ROWB, 2), lambda i: (i, 0)),
        ],
        out_specs=pl.BlockSpec((ROWB, D), lambda i: (i, 0)),
        out_shape=jax.ShapeDtypeStruct((N, D), jnp.float32),
    )(h, degp)


def _mid_body(g_ref, hs_ref, degp_ref, out_ref):
    dis = _dis_block(degp_ref[...])
    t = g_ref[0] + g_ref[1] + hs_ref[...]
    out_ref[...] = (dis * dis) * t


@jax.jit
def _tc_mid(g, hs, degp):
    return pl.pallas_call(
        _mid_body,
        grid=(N // ROWB,),
        in_specs=[
            pl.BlockSpec((2, ROWB, D), lambda i: (0, i, 0)),
            pl.BlockSpec((ROWB, D), lambda i: (i, 0)),
            pl.BlockSpec((ROWB, 2), lambda i: (i, 0)),
        ],
        out_specs=pl.BlockSpec((ROWB, D), lambda i: (i, 0)),
        out_shape=jax.ShapeDtypeStruct((N, D), jnp.float32),
    )(g, hs, degp)


def _fin_body(g_ref, hs_ref, degp_ref, out_ref):
    dis = _dis_block(degp_ref[...])
    h = dis * (g_ref[0] + g_ref[1] + hs_ref[...])
    m = jnp.max(h, axis=1, keepdims=True)
    z = h - m
    lse = jnp.log(jnp.sum(jnp.exp(z), axis=1, keepdims=True))
    out_ref[...] = z - lse


@jax.jit
def _tc_fin(g, hs, degp):
    return pl.pallas_call(
        _fin_body,
        grid=(N // ROWB,),
        in_specs=[
            pl.BlockSpec((2, ROWB, D), lambda i: (0, i, 0)),
            pl.BlockSpec((ROWB, D), lambda i: (i, 0)),
            pl.BlockSpec((ROWB, 2), lambda i: (i, 0)),
        ],
        out_specs=pl.BlockSpec((ROWB, D), lambda i: (i, 0)),
        out_shape=jax.ShapeDtypeStruct((N, D), jnp.float32),
    )(g, hs, degp)


def kernel(x, edge_index, W, b):
    row = edge_index[0]
    col = edge_index[1]
    pad = EP - E
    arp = jnp.arange(pad, dtype=jnp.int32)
    rowp = jnp.concatenate([row, arp % N])
    colp = jnp.concatenate([col, N + arp % (NPAD - N)])
    cols3 = colp.reshape(NW, NCHUNK, CHUNK)
    pk3 = (rowp | (colp << 16)).reshape(NW, NCHUNK, CHUNK)

    h = _tc_mm(x, W, b)
    degp_full = _sc_deg(cols3)
    degp = degp_full[:, :N].T

    hs1 = _tc_scale(h, degp)
    g1 = _sc_hop(hs1, pk3)
    hs2 = _tc_mid(g1, hs1, degp)
    g2 = _sc_hop(hs2, pk3)
    return _tc_fin(g2, hs2, degp)


# speedup vs baseline: 1.2242x; 1.0369x over previous
import jax
import jax.numpy as jnp
from jax import lax
from jax.experimental import pallas as pl
from jax.experimental.pallas import tpu as pltpu
from jax.experimental.pallas import tpu_sc as plsc

N = 10000
E = 320000
D = 128

NC = 2
NS = 16
NW = NC * NS

CHUNK = 128
EPW = 10240
NCHUNK = EPW // CHUNK
EP = EPW * NW
NPAD = 10240
ROWS_PER_TILE = NPAD // NS


def _mesh():
    return plsc.VectorSubcoreMesh(
        core_axis_name="c", subcore_axis_name="s", num_cores=NC, num_subcores=NS
    )


def _deg_body(cols_hbm, out_hbm, col_v, ones_v, deg_sh, zrow_v, semd):
    cid = lax.axis_index("c")
    sid = lax.axis_index("s")
    wid = sid * NC + cid

    zero16 = jnp.zeros((16,), jnp.float32)
    one16 = jnp.ones((16,), jnp.float32)
    for k in range(CHUNK // 16):
        ones_v[pl.ds(k * 16, 16)] = one16

    def zfill(i, _):
        zrow_v[pl.ds(i * 16, 16)] = zero16
        return 0
    lax.fori_loop(0, ROWS_PER_TILE // 16, zfill, 0)

    pltpu.sync_copy(zrow_v, deg_sh.at[pl.ds(sid * ROWS_PER_TILE, ROWS_PER_TILE)])
    plsc.subcore_barrier()

    pltpu.sync_copy(cols_hbm.at[wid], col_v)

    def body(j, _):
        pltpu.async_copy(ones_v, deg_sh.at[col_v.at[j]], semd, add=True)
        return 0
    lax.fori_loop(0, NCHUNK, body, 0)

    def drain(j, _):
        pltpu.make_async_copy(ones_v, deg_sh.at[col_v.at[j]], semd).wait()
        return 0
    lax.fori_loop(0, NCHUNK, drain, 0)
    plsc.subcore_barrier()

    pltpu.sync_copy(
        deg_sh.at[pl.ds(sid * ROWS_PER_TILE, ROWS_PER_TILE)],
        out_hbm.at[cid, pl.ds(sid * ROWS_PER_TILE, ROWS_PER_TILE)],
    )


@jax.jit
def _sc_deg(cols3):
    return pl.kernel(
        _deg_body,
        out_type=jax.ShapeDtypeStruct((NC, NPAD), jnp.float32),
        mesh=_mesh(),
        scratch_types=[
            pltpu.VMEM((NCHUNK, CHUNK), jnp.int32),
            pltpu.VMEM((CHUNK,), jnp.float32),
            pltpu.VMEM_SHARED((NPAD,), jnp.float32),
            pltpu.VMEM((ROWS_PER_TILE,), jnp.float32),
            pltpu.SemaphoreType.DMA,
        ],
    )(cols3)


def _hop_body(hs_hbm, pk_hbm, out_hbm,
              pk_v, row_u, col_u, gbuf0, gbuf1, zblk_v, acc_sh, sem0, sem1):
    cid = lax.axis_index("c")
    sid = lax.axis_index("s")
    wid = sid * NC + cid

    pk_cp = pltpu.async_copy(pk_hbm.at[wid], pk_v, sem1)

    zero16 = jnp.zeros((16,), jnp.float32)

    def zfill(i, _):
        for k in range(D // 16):
            zblk_v[i, pl.ds(k * 16, 16)] = zero16
        return 0
    lax.fori_loop(0, 16, zfill, 0)

    def zcopy(i, _):
        pltpu.async_copy(
            zblk_v, acc_sh.at[pl.ds(sid * ROWS_PER_TILE + i * 16, 16)], sem0
        )
        return 0
    lax.fori_loop(0, ROWS_PER_TILE // 16, zcopy, 0)

    def zdrain(i, _):
        pltpu.make_async_copy(
            zblk_v, acc_sh.at[pl.ds(sid * ROWS_PER_TILE + i * 16, 16)], sem0
        ).wait()
        return 0
    lax.fori_loop(0, ROWS_PER_TILE // 16, zdrain, 0)
    pk_cp.wait()
    plsc.subcore_barrier()

    def unpack(j, s):
        for k in range(CHUNK // 16):
            p = pk_v[j, pl.ds(k * 16, 16)]
            row_u[s, pl.ds(k * 16, 16)] = jnp.bitwise_and(p, 0xFFFF)
            col_u[s, pl.ds(k * 16, 16)] = jnp.right_shift(p, 16)
        return 0

    unpack(0, 0)
    pltpu.async_copy(hs_hbm.at[row_u.at[0]], gbuf0, sem0)

    def pair(p, _):
        unpack(2 * p + 1, 1)
        pltpu.async_copy(hs_hbm.at[row_u.at[1]], gbuf1, sem1)
        pltpu.make_async_copy(hs_hbm.at[row_u.at[0]], gbuf0, sem0).wait()
        pltpu.sync_copy(gbuf0, acc_sh.at[col_u.at[0]], add=True)

        @pl.when(p + 1 < NCHUNK // 2)
        def _():
            unpack(2 * p + 2, 0)
            pltpu.async_copy(hs_hbm.at[row_u.at[0]], gbuf0, sem0)
        pltpu.make_async_copy(hs_hbm.at[row_u.at[1]], gbuf1, sem1).wait()
        pltpu.sync_copy(gbuf1, acc_sh.at[col_u.at[1]], add=True)
        return 0

    lax.fori_loop(0, NCHUNK // 2, pair, 0)
    plsc.subcore_barrier()

    pltpu.sync_copy(
        acc_sh.at[pl.ds(sid * ROWS_PER_TILE, ROWS_PER_TILE)],
        out_hbm.at[cid, pl.ds(sid * ROWS_PER_TILE, ROWS_PER_TILE)],
    )


@jax.jit
def _sc_hop(hs, pk3):
    return pl.kernel(
        _hop_body,
        out_type=jax.ShapeDtypeStruct((NC, NPAD, D), jnp.float32),
        mesh=_mesh(),
        scratch_types=[
            pltpu.VMEM((NCHUNK, CHUNK), jnp.int32),
            pltpu.VMEM((2, CHUNK), jnp.int32),
            pltpu.VMEM((2, CHUNK), jnp.int32),
            pltpu.VMEM((CHUNK, D), jnp.float32),
            pltpu.VMEM((CHUNK, D), jnp.float32),
            pltpu.VMEM((16, D), jnp.float32),
            pltpu.VMEM_SHARED((NPAD, D), jnp.float32),
            pltpu.SemaphoreType.DMA,
            pltpu.SemaphoreType.DMA,
        ],
    )(hs, pk3)


ROWB = 2000


def _dis_block(degp_blk):
    deg = degp_blk[:, 0:1] + degp_blk[:, 1:2] + 1.0
    return lax.rsqrt(deg)


def _mm_body(x_ref, w_ref, b_ref, out_ref):
    out_ref[...] = lax.dot_general(
        x_ref[...], w_ref[...], (((1,), (1,)), ((), ())),
        preferred_element_type=jnp.float32,
    ) + b_ref[...]


def _tc_mm(x, W, b):
    return pl.pallas_call(
        _mm_body,
        grid=(N // ROWB,),
        in_specs=[
            pl.BlockSpec((ROWB, D), lambda i: (i, 0)),
            pl.BlockSpec((D, D), lambda i: (0, 0)),
            pl.BlockSpec((1, D), lambda i: (0, 0)),
        ],
        out_specs=pl.BlockSpec((ROWB, D), lambda i: (i, 0)),
        out_shape=jax.ShapeDtypeStruct((N, D), jnp.float32),
    )(x, W, b.reshape(1, D))


def _scale_body(h_ref, degp_ref, out_ref):
    dis = _dis_block(degp_ref[...])
    out_ref[...] = dis * h_ref[...]


def _tc_scale(h, degp):
    return pl.pallas_call(
        _scale_body,
        grid=(N // ROWB,),
        in_specs=[
            pl.BlockSpec((ROWB, D), lambda i: (i, 0)),
            pl.BlockSpec((ROWB, 2), lambda i: (i, 0)),
        ],
        out_specs=pl.BlockSpec((ROWB, D), lambda i: (i, 0)),
        out_shape=jax.ShapeDtypeStruct((N, D), jnp.float32),
    )(h, degp)


def _mid_body(g_ref, hs_ref, degp_ref, out_ref):
    dis = _dis_block(degp_ref[...])
    t = g_ref[0] + g_ref[1] + hs_ref[...]
    out_ref[...] = (dis * dis) * t


@jax.jit
def _tc_mid(g, hs, degp):
    return pl.pallas_call(
        _mid_body,
        grid=(N // ROWB,),
        in_specs=[
            pl.BlockSpec((2, ROWB, D), lambda i: (0, i, 0)),
            pl.BlockSpec((ROWB, D), lambda i: (i, 0)),
            pl.BlockSpec((ROWB, 2), lambda i: (i, 0)),
        ],
        out_specs=pl.BlockSpec((ROWB, D), lambda i: (i, 0)),
        out_shape=jax.ShapeDtypeStruct((N, D), jnp.float32),
    )(g, hs, degp)


def _fin_body(g_ref, hs_ref, degp_ref, out_ref):
    dis = _dis_block(degp_ref[...])
    h = dis * (g_ref[0] + g_ref[1] + hs_ref[...])
    m = jnp.max(h, axis=1, keepdims=True)
    z = h - m
    lse = jnp.log(jnp.sum(jnp.exp(z), axis=1, keepdims=True))
    out_ref[...] = z - lse


@jax.jit
def _tc_fin(g, hs, degp):
    return pl.pallas_call(
        _fin_body,
        grid=(N // ROWB,),
        in_specs=[
            pl.BlockSpec((2, ROWB, D), lambda i: (0, i, 0)),
            pl.BlockSpec((ROWB, D), lambda i: (i, 0)),
            pl.BlockSpec((ROWB, 2), lambda i: (i, 0)),
        ],
        out_specs=pl.BlockSpec((ROWB, D), lambda i: (i, 0)),
        out_shape=jax.ShapeDtypeStruct((N, D), jnp.float32),
    )(g, hs, degp)


def kernel(x, edge_index, W, b):
    row = edge_index[0]
    col = edge_index[1]
    pad = EP - E
    arp = jnp.arange(pad, dtype=jnp.int32)
    rowp = jnp.concatenate([row, arp % N])
    colp = jnp.concatenate([col, N + arp % (NPAD - N)])
    cols3 = colp.reshape(NW, NCHUNK, CHUNK)
    pk3 = (rowp | (colp << 16)).reshape(NW, NCHUNK, CHUNK)

    h = _tc_mm(x, W, b)
    degp_full = _sc_deg(cols3)
    degp = degp_full[:, :N].T

    hs1 = _tc_scale(h, degp)
    g1 = _sc_hop(hs1, pk3)
    hs2 = _tc_mid(g1, hs1, degp)
    g2 = _sc_hop(hs2, pk3)
    return _tc_fin(g2, hs2, degp)
